# baseline (speedup 1.0000x reference)
import functools

import jax
import jax.numpy as jnp
from jax import lax
from jax.experimental import pallas as pl
from jax.experimental.pallas import tpu as pltpu
from jax.experimental.pallas import tpu_sc as plsc

N = 10000
NP = 10240
E = 160000
D = 256
HALF = 128

NC = 2
NS = 16
LANES = 16

CHUNK = 96
NCHUNK = 105
EDGES_PER_TILE = CHUNK * NCHUNK
E_PAD = EDGES_PER_TILE * NS
ROWS_PER_TILE = NP // NS
ROW_STAGE = 80

_mesh = plsc.VectorSubcoreMesh(core_axis_name="c", subcore_axis_name="s")
_sc_params = pltpu.CompilerParams(use_tc_tiling_on_sc=False)


def _load_idx_chunk(dst_ref, src_ref, base):
    for k in range(CHUNK // LANES):
        start = pl.multiple_of(base + k * LANES, LANES)
        dst_ref[pl.ds(k * LANES, LANES)] = src_ref[pl.ds(start, LANES)]


def _spmm_body(src_hbm, dst_hbm, m0_hbm, m1_hbm, out0_hbm, out1_hbm,
               acc_sh, sidx_all, didx_all,
               sidx_a, didx_a, rows_a, sem_a,
               sidx_b, didx_b, rows_b, sem_b):
    c = lax.axis_index("c")
    s = lax.axis_index("s")

    def work(m_hbm, out_hbm):
        ebase = s * EDGES_PER_TILE
        pltpu.sync_copy(src_hbm.at[pl.ds(ebase, EDGES_PER_TILE)], sidx_all)
        pltpu.sync_copy(dst_hbm.at[pl.ds(ebase, EDGES_PER_TILE)], didx_all)
        nstage = ROWS_PER_TILE // ROW_STAGE
        bufs = [(rows_a, sem_a), (rows_b, sem_b)]

        def stg(j):
            return pl.ds(s * ROWS_PER_TILE + j * ROW_STAGE, ROW_STAGE)

        pltpu.async_copy(m_hbm.at[stg(0)], rows_a.at[pl.ds(0, ROW_STAGE)],
                         sem_a)
        for j in range(nstage):
            buf, sem = bufs[j % 2]
            nbuf, nsem = bufs[(j + 1) % 2]
            if j + 1 < nstage:
                pltpu.async_copy(m_hbm.at[stg(j + 1)],
                                 nbuf.at[pl.ds(0, ROW_STAGE)], nsem)
            pltpu.make_async_copy(m_hbm.at[stg(j)],
                                  buf.at[pl.ds(0, ROW_STAGE)], sem).wait()
            pltpu.sync_copy(buf.at[pl.ds(0, ROW_STAGE)], acc_sh.at[stg(j)])
        plsc.subcore_barrier()

        def start_gather(idx_ref, rows_ref, sem, chunk):
            _load_idx_chunk(idx_ref, sidx_all, chunk * CHUNK)
            pltpu.async_copy(m_hbm.at[idx_ref], rows_ref, sem)

        start_gather(sidx_a, rows_a, sem_a, 0)

        def pair_body(j, _):
            i0 = j * 2
            start_gather(sidx_b, rows_b, sem_b, i0 + 1)
            pltpu.make_async_copy(m_hbm.at[sidx_a], rows_a, sem_a).wait()
            _load_idx_chunk(didx_a, didx_all, i0 * CHUNK)
            pltpu.sync_copy(rows_a, acc_sh.at[didx_a], add=True)
            start_gather(sidx_a, rows_a, sem_a, i0 + 2)
            pltpu.make_async_copy(m_hbm.at[sidx_b], rows_b, sem_b).wait()
            _load_idx_chunk(didx_b, didx_all, (i0 + 1) * CHUNK)
            pltpu.sync_copy(rows_b, acc_sh.at[didx_b], add=True)
            return 0

        lax.fori_loop(0, (NCHUNK - 1) // 2, pair_body, 0)
        pltpu.make_async_copy(m_hbm.at[sidx_a], rows_a, sem_a).wait()
        _load_idx_chunk(didx_a, didx_all, (NCHUNK - 1) * CHUNK)
        pltpu.sync_copy(rows_a, acc_sh.at[didx_a], add=True)
        plsc.subcore_barrier()
        for j in range(nstage):
            buf, sem = bufs[j % 2]
            if j >= 2:
                pltpu.make_async_copy(buf.at[pl.ds(0, ROW_STAGE)],
                                      out_hbm.at[stg(j - 2)], sem).wait()
            pltpu.sync_copy(acc_sh.at[stg(j)], buf.at[pl.ds(0, ROW_STAGE)])
            pltpu.async_copy(buf.at[pl.ds(0, ROW_STAGE)], out_hbm.at[stg(j)],
                             sem)
        pltpu.make_async_copy(rows_a.at[pl.ds(0, ROW_STAGE)],
                              out_hbm.at[stg(nstage - 2)], sem_a).wait()
        pltpu.make_async_copy(rows_b.at[pl.ds(0, ROW_STAGE)],
                              out_hbm.at[stg(nstage - 1)], sem_b).wait()

    @pl.when(c == 0)
    def _():
        work(m0_hbm, out0_hbm)

    @pl.when(c == 1)
    def _():
        work(m1_hbm, out1_hbm)


@functools.partial(jax.jit, donate_argnums=())
def _sc_spmm(src, dst, m0, m1):
    return pl.kernel(
        _spmm_body,
        out_type=(
            jax.ShapeDtypeStruct((NP, HALF), jnp.float32),
            jax.ShapeDtypeStruct((NP, HALF), jnp.float32),
        ),
        mesh=_mesh,
        scratch_types=[
            pltpu.VMEM_SHARED((NP, HALF), jnp.float32),
            pltpu.VMEM((EDGES_PER_TILE,), jnp.int32),
            pltpu.VMEM((EDGES_PER_TILE,), jnp.int32),
            pltpu.VMEM((CHUNK,), jnp.int32),
            pltpu.VMEM((CHUNK,), jnp.int32),
            pltpu.VMEM((CHUNK, HALF), jnp.float32),
            pltpu.SemaphoreType.DMA,
            pltpu.VMEM((CHUNK,), jnp.int32),
            pltpu.VMEM((CHUNK,), jnp.int32),
            pltpu.VMEM((CHUNK, HALF), jnp.float32),
            pltpu.SemaphoreType.DMA,
        ],
        compiler_params=_sc_params,
    )(src, dst, m0, m1)


def _deg_body(dst_hbm, degp_hbm, acc_sh, stage_v, didx_all, didx_c,
              ones_v, zeros_v):
    c = lax.axis_index("c")
    s = lax.axis_index("s")

    def fill(i, _):
        ones_v[i] = jnp.ones((LANES,), jnp.float32)
        return 0

    lax.fori_loop(0, CHUNK, fill, 0)

    def zfill(i, _):
        zeros_v[i] = jnp.zeros((LANES,), jnp.float32)
        return 0

    lax.fori_loop(0, ROWS_PER_TILE, zfill, 0)
    pltpu.sync_copy(zeros_v, acc_sh.at[pl.ds(s * ROWS_PER_TILE, ROWS_PER_TILE)])
    pltpu.sync_copy(dst_hbm.at[pl.ds(s * EDGES_PER_TILE, EDGES_PER_TILE)],
                    didx_all)
    plsc.subcore_barrier()

    def chunk_body(i, _):
        _load_idx_chunk(didx_c, didx_all, i * CHUNK)
        pltpu.sync_copy(ones_v, acc_sh.at[didx_c], add=True)
        return 0

    lax.fori_loop(0, NCHUNK, chunk_body, 0)
    plsc.subcore_barrier()
    rbase = s * ROWS_PER_TILE
    pltpu.sync_copy(acc_sh.at[pl.ds(rbase, ROWS_PER_TILE)], stage_v)
    pltpu.sync_copy(stage_v, degp_hbm.at[c].at[pl.ds(rbase, ROWS_PER_TILE)])


@jax.jit
def _sc_degree(dst):
    return pl.kernel(
        _deg_body,
        out_type=jax.ShapeDtypeStruct((NC, NP, LANES), jnp.float32),
        mesh=_mesh,
        scratch_types=[
            pltpu.VMEM_SHARED((NP, LANES), jnp.float32),
            pltpu.VMEM((ROWS_PER_TILE, LANES), jnp.float32),
            pltpu.VMEM((EDGES_PER_TILE,), jnp.int32),
            pltpu.VMEM((CHUNK,), jnp.int32),
            pltpu.VMEM((CHUNK, LANES), jnp.float32),
            pltpu.VMEM((ROWS_PER_TILE, LANES), jnp.float32),
        ],
        compiler_params=_sc_params,
    )(dst)


_RB = 1024
_GRID = NP // _RB


def _dinv_from_degp(dg_blk):
    deg = jnp.sum(dg_blk, axis=(0, 2)) * (1.0 / 32.0) + 1.0
    return lax.rsqrt(jnp.maximum(deg, 1.0))


def _pre_body(dg_ref, x_ref, w_ref, m0_ref, m1_ref):
    dinv = _dinv_from_degp(dg_ref[...])
    h = jnp.dot(x_ref[...], w_ref[...], preferred_element_type=jnp.float32)
    m = h * dinv[:, None]
    m0_ref[...] = m[:, :HALF]
    m1_ref[...] = m[:, HALF:]


def _mid_body(dg_ref, x_ref, a0_ref, a1_ref, w_ref, xn_ref, m0_ref, m1_ref):
    dinv = _dinv_from_degp(dg_ref[...])
    g = jnp.concatenate([a0_ref[...], a1_ref[...]], axis=1) * dinv[:, None]
    xn = x_ref[...] + jnp.maximum(g, 0.0)
    xn_ref[...] = xn
    h = jnp.dot(xn, w_ref[...], preferred_element_type=jnp.float32)
    m = h * dinv[:, None]
    m0_ref[...] = m[:, :HALF]
    m1_ref[...] = m[:, HALF:]


def _post_body(dg_ref, x_ref, a0_ref, a1_ref, out_ref):
    dinv = _dinv_from_degp(dg_ref[...])
    g = jnp.concatenate([a0_ref[...], a1_ref[...]], axis=1) * dinv[:, None]
    xn = x_ref[...] + jnp.maximum(g, 0.0)
    nrm = jnp.sqrt(jnp.sum(xn * xn, axis=1, keepdims=True))
    out_ref[...] = xn / jnp.maximum(nrm, 1e-12)


_dg_spec = pl.BlockSpec((NC, _RB, LANES), lambda i: (0, i, 0))
_x_spec = pl.BlockSpec((_RB, D), lambda i: (i, 0))
_h_spec = pl.BlockSpec((_RB, HALF), lambda i: (i, 0))
_w_spec = pl.BlockSpec((D, D), lambda i: (0, 0))


def _tc_pre(degp, x, w):
    return pl.pallas_call(
        _pre_body,
        grid=(_GRID,),
        in_specs=[_dg_spec, _x_spec, _w_spec],
        out_specs=[_h_spec, _h_spec],
        out_shape=[jax.ShapeDtypeStruct((NP, HALF), jnp.float32)] * 2,
    )(degp, x, w)


def _tc_mid(degp, x, a0, a1, w):
    return pl.pallas_call(
        _mid_body,
        grid=(_GRID,),
        in_specs=[_dg_spec, _x_spec, _h_spec, _h_spec, _w_spec],
        out_specs=[_x_spec, _h_spec, _h_spec],
        out_shape=[jax.ShapeDtypeStruct((NP, D), jnp.float32),
                   jax.ShapeDtypeStruct((NP, HALF), jnp.float32),
                   jax.ShapeDtypeStruct((NP, HALF), jnp.float32)],
    )(degp, x, a0, a1, w)


def _tc_post(degp, x, a0, a1):
    return pl.pallas_call(
        _post_body,
        grid=(_GRID,),
        in_specs=[_dg_spec, _x_spec, _h_spec, _h_spec],
        out_specs=_x_spec,
        out_shape=jax.ShapeDtypeStruct((NP, D), jnp.float32),
    )(degp, x, a0, a1)


def kernel(x, edge_index, W0, W1, W2):
    pad_e = jnp.full((E_PAD - E,), NP - 1, jnp.int32)
    src = jnp.concatenate([edge_index[0], pad_e])
    dst = jnp.concatenate([edge_index[1], pad_e])
    xp = jnp.pad(x, ((0, NP - N), (0, 0)))
    degp = _sc_degree(dst)
    m0, m1 = _tc_pre(degp, xp, W0)
    a0, a1 = _sc_spmm(src, dst, m0, m1)
    x1, m0, m1 = _tc_mid(degp, xp, a0, a1, W1)
    a0, a1 = _sc_spmm(src, dst, m0, m1)
    x2, m0, m1 = _tc_mid(degp, x1, a0, a1, W2)
    a0, a1 = _sc_spmm(src, dst, m0, m1)
    return _tc_post(degp, x2, a0, a1)[:N]


# speedup vs baseline: 1.3624x; 1.3624x over previous
import functools

import jax
import jax.numpy as jnp
from jax import lax
from jax.experimental import pallas as pl
from jax.experimental.pallas import tpu as pltpu
from jax.experimental.pallas import tpu_sc as plsc

N = 10000
NP = 10240
E = 160000
D = 256
HALF = 128

NC = 2
NS = 16
LANES = 16

CHUNK = 96
NCHUNK = 105
EDGES_PER_TILE = CHUNK * NCHUNK
E_PAD = EDGES_PER_TILE * NS
ROWS_PER_TILE = NP // NS
ROW_STAGE = 80

_mesh = plsc.VectorSubcoreMesh(core_axis_name="c", subcore_axis_name="s")
_sc_params = pltpu.CompilerParams(use_tc_tiling_on_sc=False)


def _load_idx_chunk(dst_ref, src_ref, base):
    for k in range(CHUNK // LANES):
        start = pl.multiple_of(base + k * LANES, LANES)
        dst_ref[pl.ds(k * LANES, LANES)] = src_ref[pl.ds(start, LANES)]


def _spmm_body(src_hbm, dst_hbm, m0_hbm, m1_hbm, out0_hbm, out1_hbm,
               acc_sh, sidx_all, didx_all,
               sidx_a, didx_a, rows_a, sem_a,
               sidx_b, didx_b, rows_b, sem_b):
    c = lax.axis_index("c")
    s = lax.axis_index("s")

    def work(m_hbm, out_hbm):
        ebase = s * EDGES_PER_TILE
        pltpu.sync_copy(src_hbm.at[pl.ds(ebase, EDGES_PER_TILE)], sidx_all)
        pltpu.sync_copy(dst_hbm.at[pl.ds(ebase, EDGES_PER_TILE)], didx_all)
        nstage = ROWS_PER_TILE // ROW_STAGE
        bufs = [(rows_a, sem_a), (rows_b, sem_b)]

        def stg(j):
            return pl.ds(s * ROWS_PER_TILE + j * ROW_STAGE, ROW_STAGE)

        pltpu.async_copy(m_hbm.at[stg(0)], rows_a.at[pl.ds(0, ROW_STAGE)],
                         sem_a)
        for j in range(nstage):
            buf, sem = bufs[j % 2]
            nbuf, nsem = bufs[(j + 1) % 2]
            if j + 1 < nstage:
                pltpu.async_copy(m_hbm.at[stg(j + 1)],
                                 nbuf.at[pl.ds(0, ROW_STAGE)], nsem)
            pltpu.make_async_copy(m_hbm.at[stg(j)],
                                  buf.at[pl.ds(0, ROW_STAGE)], sem).wait()
            pltpu.sync_copy(buf.at[pl.ds(0, ROW_STAGE)], acc_sh.at[stg(j)])
        plsc.subcore_barrier()

        def start_gather(idx_ref, rows_ref, sem, chunk):
            _load_idx_chunk(idx_ref, sidx_all, chunk * CHUNK)
            pltpu.async_copy(m_hbm.at[idx_ref], rows_ref, sem)

        start_gather(sidx_a, rows_a, sem_a, 0)

        def pair_body(j, _):
            i0 = j * 2
            start_gather(sidx_b, rows_b, sem_b, i0 + 1)
            pltpu.make_async_copy(m_hbm.at[sidx_a], rows_a, sem_a).wait()
            _load_idx_chunk(didx_a, didx_all, i0 * CHUNK)
            pltpu.sync_copy(rows_a, acc_sh.at[didx_a], add=True)
            start_gather(sidx_a, rows_a, sem_a, i0 + 2)
            pltpu.make_async_copy(m_hbm.at[sidx_b], rows_b, sem_b).wait()
            _load_idx_chunk(didx_b, didx_all, (i0 + 1) * CHUNK)
            pltpu.sync_copy(rows_b, acc_sh.at[didx_b], add=True)
            return 0

        lax.fori_loop(0, (NCHUNK - 1) // 2, pair_body, 0)
        pltpu.make_async_copy(m_hbm.at[sidx_a], rows_a, sem_a).wait()
        _load_idx_chunk(didx_a, didx_all, (NCHUNK - 1) * CHUNK)
        pltpu.sync_copy(rows_a, acc_sh.at[didx_a], add=True)
        plsc.subcore_barrier()
        for j in range(nstage):
            buf, sem = bufs[j % 2]
            if j >= 2:
                pltpu.make_async_copy(buf.at[pl.ds(0, ROW_STAGE)],
                                      out_hbm.at[stg(j - 2)], sem).wait()
            pltpu.sync_copy(acc_sh.at[stg(j)], buf.at[pl.ds(0, ROW_STAGE)])
            pltpu.async_copy(buf.at[pl.ds(0, ROW_STAGE)], out_hbm.at[stg(j)],
                             sem)
        pltpu.make_async_copy(rows_a.at[pl.ds(0, ROW_STAGE)],
                              out_hbm.at[stg(nstage - 2)], sem_a).wait()
        pltpu.make_async_copy(rows_b.at[pl.ds(0, ROW_STAGE)],
                              out_hbm.at[stg(nstage - 1)], sem_b).wait()

    @pl.when(c == 0)
    def _():
        work(m0_hbm, out0_hbm)

    @pl.when(c == 1)
    def _():
        work(m1_hbm, out1_hbm)


@functools.partial(jax.jit, donate_argnums=())
def _sc_spmm(src, dst, m0, m1):
    return pl.kernel(
        _spmm_body,
        out_type=(
            jax.ShapeDtypeStruct((NP, HALF), jnp.float32),
            jax.ShapeDtypeStruct((NP, HALF), jnp.float32),
        ),
        mesh=_mesh,
        scratch_types=[
            pltpu.VMEM_SHARED((NP, HALF), jnp.float32),
            pltpu.VMEM((EDGES_PER_TILE,), jnp.int32),
            pltpu.VMEM((EDGES_PER_TILE,), jnp.int32),
            pltpu.VMEM((CHUNK,), jnp.int32),
            pltpu.VMEM((CHUNK,), jnp.int32),
            pltpu.VMEM((CHUNK, HALF), jnp.float32),
            pltpu.SemaphoreType.DMA,
            pltpu.VMEM((CHUNK,), jnp.int32),
            pltpu.VMEM((CHUNK,), jnp.int32),
            pltpu.VMEM((CHUNK, HALF), jnp.float32),
            pltpu.SemaphoreType.DMA,
        ],
        compiler_params=_sc_params,
    )(src, dst, m0, m1)


def _deg_body(dst_hbm, degp_hbm, acc_sh, stage_v, didx_all, didx_c,
              ones_v, zeros_v):
    c = lax.axis_index("c")
    s = lax.axis_index("s")

    def fill(i, _):
        ones_v[i] = jnp.ones((LANES,), jnp.float32)
        return 0

    lax.fori_loop(0, CHUNK, fill, 0)

    def zfill(i, _):
        zeros_v[i] = jnp.zeros((LANES,), jnp.float32)
        return 0

    lax.fori_loop(0, ROWS_PER_TILE, zfill, 0)
    pltpu.sync_copy(zeros_v, acc_sh.at[pl.ds(s * ROWS_PER_TILE, ROWS_PER_TILE)])
    pltpu.sync_copy(dst_hbm.at[pl.ds(s * EDGES_PER_TILE, EDGES_PER_TILE)],
                    didx_all)
    plsc.subcore_barrier()

    def chunk_body(i, _):
        _load_idx_chunk(didx_c, didx_all, i * CHUNK)
        pltpu.sync_copy(ones_v, acc_sh.at[didx_c], add=True)
        return 0

    lax.fori_loop(0, NCHUNK, chunk_body, 0)
    plsc.subcore_barrier()
    rbase = s * ROWS_PER_TILE
    pltpu.sync_copy(acc_sh.at[pl.ds(rbase, ROWS_PER_TILE)], stage_v)
    pltpu.sync_copy(stage_v, degp_hbm.at[c].at[pl.ds(rbase, ROWS_PER_TILE)])


@jax.jit
def _sc_degree(dst):
    return pl.kernel(
        _deg_body,
        out_type=jax.ShapeDtypeStruct((NC, NP, LANES), jnp.float32),
        mesh=_mesh,
        scratch_types=[
            pltpu.VMEM_SHARED((NP, LANES), jnp.float32),
            pltpu.VMEM((ROWS_PER_TILE, LANES), jnp.float32),
            pltpu.VMEM((EDGES_PER_TILE,), jnp.int32),
            pltpu.VMEM((CHUNK,), jnp.int32),
            pltpu.VMEM((CHUNK, LANES), jnp.float32),
            pltpu.VMEM((ROWS_PER_TILE, LANES), jnp.float32),
        ],
        compiler_params=_sc_params,
    )(dst)


_RB = 1024
_GRID = NP // _RB


def _dinv_from_degp(dg_blk):
    deg = jnp.sum(dg_blk, axis=(0, 2)) * (1.0 / 32.0) + 1.0
    return lax.rsqrt(jnp.maximum(deg, 1.0))


def _pre_body(dg_ref, x_ref, w_ref, m0_ref, m1_ref):
    dinv = _dinv_from_degp(dg_ref[...])
    h = jnp.dot(x_ref[...], w_ref[...], preferred_element_type=jnp.float32)
    m = h * dinv[:, None]
    m0_ref[...] = m[:, :HALF]
    m1_ref[...] = m[:, HALF:]


def _mid_body(dg_ref, x_ref, a0_ref, a1_ref, w_ref, xn_ref, m0_ref, m1_ref):
    dinv = _dinv_from_degp(dg_ref[...])
    g = jnp.concatenate([a0_ref[...], a1_ref[...]], axis=1) * dinv[:, None]
    xn = x_ref[...] + jnp.maximum(g, 0.0)
    xn_ref[...] = xn
    h = jnp.dot(xn, w_ref[...], preferred_element_type=jnp.float32)
    m = h * dinv[:, None]
    m0_ref[...] = m[:, :HALF]
    m1_ref[...] = m[:, HALF:]


def _post_body(dg_ref, x_ref, a0_ref, a1_ref, out_ref):
    dinv = _dinv_from_degp(dg_ref[...])
    g = jnp.concatenate([a0_ref[...], a1_ref[...]], axis=1) * dinv[:, None]
    xn = x_ref[...] + jnp.maximum(g, 0.0)
    nrm = jnp.sqrt(jnp.sum(xn * xn, axis=1, keepdims=True))
    out_ref[...] = xn / jnp.maximum(nrm, 1e-12)


_dg_spec = pl.BlockSpec((NC, _RB, LANES), lambda i: (0, i, 0))
_x_spec = pl.BlockSpec((_RB, D), lambda i: (i, 0))
_h_spec = pl.BlockSpec((_RB, HALF), lambda i: (i, 0))
_w_spec = pl.BlockSpec((D, D), lambda i: (0, 0))


def _tc_pre(degp, x, w):
    return pl.pallas_call(
        _pre_body,
        grid=(_GRID,),
        in_specs=[_dg_spec, _x_spec, _w_spec],
        out_specs=[_h_spec, _h_spec],
        out_shape=[jax.ShapeDtypeStruct((NP, HALF), jnp.float32)] * 2,
    )(degp, x, w)


def _tc_mid(degp, x, a0, a1, w):
    return pl.pallas_call(
        _mid_body,
        grid=(_GRID,),
        in_specs=[_dg_spec, _x_spec, _h_spec, _h_spec, _w_spec],
        out_specs=[_x_spec, _h_spec, _h_spec],
        out_shape=[jax.ShapeDtypeStruct((NP, D), jnp.float32),
                   jax.ShapeDtypeStruct((NP, HALF), jnp.float32),
                   jax.ShapeDtypeStruct((NP, HALF), jnp.float32)],
    )(degp, x, a0, a1, w)


def _tc_post(degp, x, a0, a1):
    return pl.pallas_call(
        _post_body,
        grid=(_GRID,),
        in_specs=[_dg_spec, _x_spec, _h_spec, _h_spec],
        out_specs=_x_spec,
        out_shape=jax.ShapeDtypeStruct((NP, D), jnp.float32),
    )(degp, x, a0, a1)


def kernel(x, edge_index, W0, W1, W2):
    pad_e = N + (jnp.arange(E_PAD - E, dtype=jnp.int32) % (NP - N))
    src = jnp.concatenate([edge_index[0], pad_e])
    dst = jnp.concatenate([edge_index[1], pad_e])
    xp = jnp.pad(x, ((0, NP - N), (0, 0)))
    degp = _sc_degree(dst)
    m0, m1 = _tc_pre(degp, xp, W0)
    a0, a1 = _sc_spmm(src, dst, m0, m1)
    x1, m0, m1 = _tc_mid(degp, xp, a0, a1, W1)
    a0, a1 = _sc_spmm(src, dst, m0, m1)
    x2, m0, m1 = _tc_mid(degp, x1, a0, a1, W2)
    a0, a1 = _sc_spmm(src, dst, m0, m1)
    return _tc_post(degp, x2, a0, a1)[:N]


# speedup vs baseline: 1.3794x; 1.0125x over previous
import functools

import jax
import jax.numpy as jnp
from jax import lax
from jax.experimental import pallas as pl
from jax.experimental.pallas import tpu as pltpu
from jax.experimental.pallas import tpu_sc as plsc

N = 10000
NP = 10240
E = 160000
D = 256
HALF = 128

NC = 2
NS = 16
LANES = 16

CHUNK = 96
NCHUNK = 105
EDGES_PER_TILE = CHUNK * NCHUNK
E_PAD = EDGES_PER_TILE * NS
ROWS_PER_TILE = NP // NS
ROW_STAGE = 80

_mesh = plsc.VectorSubcoreMesh(core_axis_name="c", subcore_axis_name="s")
_sc_params = pltpu.CompilerParams(use_tc_tiling_on_sc=False)


def _load_idx_chunk(dst_ref, src_ref, base):
    for k in range(CHUNK // LANES):
        start = pl.multiple_of(base + k * LANES, LANES)
        dst_ref[pl.ds(k * LANES, LANES)] = src_ref[pl.ds(start, LANES)]


def _spmm_body(src_hbm, dst_hbm, m0_hbm, m1_hbm, out0_hbm, out1_hbm,
               acc_sh, sidx_all, didx_all,
               sidx_a, didx_a, rows_a, sem_a,
               sidx_b, didx_b, rows_b, sem_b):
    c = lax.axis_index("c")
    s = lax.axis_index("s")

    def work(m_hbm, out_hbm):
        ebase = s * EDGES_PER_TILE
        pltpu.sync_copy(src_hbm.at[pl.ds(ebase, EDGES_PER_TILE)], sidx_all)
        pltpu.sync_copy(dst_hbm.at[pl.ds(ebase, EDGES_PER_TILE)], didx_all)
        nstage = ROWS_PER_TILE // ROW_STAGE
        bufs = [(rows_a, sem_a), (rows_b, sem_b)]

        def stg(j):
            return pl.ds(s * ROWS_PER_TILE + j * ROW_STAGE, ROW_STAGE)

        pltpu.async_copy(m_hbm.at[stg(0)], rows_a.at[pl.ds(0, ROW_STAGE)],
                         sem_a)
        for j in range(nstage):
            buf, sem = bufs[j % 2]
            nbuf, nsem = bufs[(j + 1) % 2]
            if j + 1 < nstage:
                pltpu.async_copy(m_hbm.at[stg(j + 1)],
                                 nbuf.at[pl.ds(0, ROW_STAGE)], nsem)
            pltpu.make_async_copy(m_hbm.at[stg(j)],
                                  buf.at[pl.ds(0, ROW_STAGE)], sem).wait()
            pltpu.sync_copy(buf.at[pl.ds(0, ROW_STAGE)], acc_sh.at[stg(j)])
        plsc.subcore_barrier()

        def start_gather(idx_ref, rows_ref, sem, chunk):
            _load_idx_chunk(idx_ref, sidx_all, chunk * CHUNK)
            pltpu.async_copy(m_hbm.at[idx_ref], rows_ref, sem)

        start_gather(sidx_a, rows_a, sem_a, 0)

        def pair_body(j, _):
            i0 = j * 2
            start_gather(sidx_b, rows_b, sem_b, i0 + 1)
            pltpu.make_async_copy(m_hbm.at[sidx_a], rows_a, sem_a).wait()
            _load_idx_chunk(didx_a, didx_all, i0 * CHUNK)
            pltpu.sync_copy(rows_a, acc_sh.at[didx_a], add=True)
            start_gather(sidx_a, rows_a, sem_a, i0 + 2)
            pltpu.make_async_copy(m_hbm.at[sidx_b], rows_b, sem_b).wait()
            _load_idx_chunk(didx_b, didx_all, (i0 + 1) * CHUNK)
            pltpu.sync_copy(rows_b, acc_sh.at[didx_b], add=True)
            return 0

        lax.fori_loop(0, (NCHUNK - 1) // 2, pair_body, 0)
        pltpu.make_async_copy(m_hbm.at[sidx_a], rows_a, sem_a).wait()
        _load_idx_chunk(didx_a, didx_all, (NCHUNK - 1) * CHUNK)
        pltpu.sync_copy(rows_a, acc_sh.at[didx_a], add=True)
        plsc.subcore_barrier()
        for j in range(nstage):
            buf, sem = bufs[j % 2]
            if j >= 2:
                pltpu.make_async_copy(buf.at[pl.ds(0, ROW_STAGE)],
                                      out_hbm.at[stg(j - 2)], sem).wait()
            pltpu.sync_copy(acc_sh.at[stg(j)], buf.at[pl.ds(0, ROW_STAGE)])
            pltpu.async_copy(buf.at[pl.ds(0, ROW_STAGE)], out_hbm.at[stg(j)],
                             sem)
        pltpu.make_async_copy(rows_a.at[pl.ds(0, ROW_STAGE)],
                              out_hbm.at[stg(nstage - 2)], sem_a).wait()
        pltpu.make_async_copy(rows_b.at[pl.ds(0, ROW_STAGE)],
                              out_hbm.at[stg(nstage - 1)], sem_b).wait()

    @pl.when(c == 0)
    def _():
        work(m0_hbm, out0_hbm)

    @pl.when(c == 1)
    def _():
        work(m1_hbm, out1_hbm)


@functools.partial(jax.jit, donate_argnums=())
def _sc_spmm(src, dst, m0, m1):
    return pl.kernel(
        _spmm_body,
        out_type=(
            jax.ShapeDtypeStruct((NP, HALF), jnp.float32),
            jax.ShapeDtypeStruct((NP, HALF), jnp.float32),
        ),
        mesh=_mesh,
        scratch_types=[
            pltpu.VMEM_SHARED((NP, HALF), jnp.float32),
            pltpu.VMEM((EDGES_PER_TILE,), jnp.int32),
            pltpu.VMEM((EDGES_PER_TILE,), jnp.int32),
            pltpu.VMEM((CHUNK,), jnp.int32),
            pltpu.VMEM((CHUNK,), jnp.int32),
            pltpu.VMEM((CHUNK, HALF), jnp.float32),
            pltpu.SemaphoreType.DMA,
            pltpu.VMEM((CHUNK,), jnp.int32),
            pltpu.VMEM((CHUNK,), jnp.int32),
            pltpu.VMEM((CHUNK, HALF), jnp.float32),
            pltpu.SemaphoreType.DMA,
        ],
        compiler_params=_sc_params,
    )(src, dst, m0, m1)


def _deg_body(dst_hbm, degp_hbm, acc_sh, stage_v, didx_all, didx_c,
              ones_v, zeros_v, didx_d, sem_c, sem_d):
    c = lax.axis_index("c")
    s = lax.axis_index("s")

    def fill(i, _):
        ones_v[i] = jnp.ones((LANES,), jnp.float32)
        return 0

    lax.fori_loop(0, CHUNK, fill, 0)

    def zfill(i, _):
        zeros_v[i] = jnp.zeros((LANES,), jnp.float32)
        return 0

    lax.fori_loop(0, ROWS_PER_TILE, zfill, 0)
    pltpu.sync_copy(zeros_v, acc_sh.at[pl.ds(s * ROWS_PER_TILE, ROWS_PER_TILE)])
    pltpu.sync_copy(dst_hbm.at[pl.ds(s * EDGES_PER_TILE, EDGES_PER_TILE)],
                    didx_all)
    plsc.subcore_barrier()

    def issue(didx_ref, sem, chunk):
        _load_idx_chunk(didx_ref, didx_all, chunk * CHUNK)
        pltpu.async_copy(ones_v, acc_sh.at[didx_ref], sem, add=True)

    def drain(didx_ref, sem):
        pltpu.make_async_copy(ones_v, acc_sh.at[didx_ref], sem).wait()

    issue(didx_c, sem_c, 0)
    issue(didx_d, sem_d, 1)

    def pair_body(j, _):
        drain(didx_c, sem_c)
        issue(didx_c, sem_c, 2 * j)
        drain(didx_d, sem_d)
        issue(didx_d, sem_d, 2 * j + 1)
        return 0

    lax.fori_loop(1, (NCHUNK - 1) // 2, pair_body, 0)
    drain(didx_c, sem_c)
    issue(didx_c, sem_c, NCHUNK - 1)
    drain(didx_d, sem_d)
    drain(didx_c, sem_c)
    plsc.subcore_barrier()
    rbase = s * ROWS_PER_TILE
    pltpu.sync_copy(acc_sh.at[pl.ds(rbase, ROWS_PER_TILE)], stage_v)
    pltpu.sync_copy(stage_v, degp_hbm.at[c].at[pl.ds(rbase, ROWS_PER_TILE)])


@jax.jit
def _sc_degree(dst):
    return pl.kernel(
        _deg_body,
        out_type=jax.ShapeDtypeStruct((NC, NP, LANES), jnp.float32),
        mesh=_mesh,
        scratch_types=[
            pltpu.VMEM_SHARED((NP, LANES), jnp.float32),
            pltpu.VMEM((ROWS_PER_TILE, LANES), jnp.float32),
            pltpu.VMEM((EDGES_PER_TILE,), jnp.int32),
            pltpu.VMEM((CHUNK,), jnp.int32),
            pltpu.VMEM((CHUNK, LANES), jnp.float32),
            pltpu.VMEM((ROWS_PER_TILE, LANES), jnp.float32),
            pltpu.VMEM((CHUNK,), jnp.int32),
            pltpu.SemaphoreType.DMA,
            pltpu.SemaphoreType.DMA,
        ],
        compiler_params=_sc_params,
    )(dst)


_RB = 1024
_GRID = NP // _RB


def _dinv_from_degp(dg_blk):
    deg = jnp.sum(dg_blk, axis=(0, 2)) * (1.0 / 32.0) + 1.0
    return lax.rsqrt(jnp.maximum(deg, 1.0))


def _pre_body(dg_ref, x_ref, w_ref, m0_ref, m1_ref):
    dinv = _dinv_from_degp(dg_ref[...])
    h = jnp.dot(x_ref[...], w_ref[...], preferred_element_type=jnp.float32)
    m = h * dinv[:, None]
    m0_ref[...] = m[:, :HALF]
    m1_ref[...] = m[:, HALF:]


def _mid_body(dg_ref, x_ref, a0_ref, a1_ref, w_ref, xn_ref, m0_ref, m1_ref):
    dinv = _dinv_from_degp(dg_ref[...])
    g = jnp.concatenate([a0_ref[...], a1_ref[...]], axis=1) * dinv[:, None]
    xn = x_ref[...] + jnp.maximum(g, 0.0)
    xn_ref[...] = xn
    h = jnp.dot(xn, w_ref[...], preferred_element_type=jnp.float32)
    m = h * dinv[:, None]
    m0_ref[...] = m[:, :HALF]
    m1_ref[...] = m[:, HALF:]


def _post_body(dg_ref, x_ref, a0_ref, a1_ref, out_ref):
    dinv = _dinv_from_degp(dg_ref[...])
    g = jnp.concatenate([a0_ref[...], a1_ref[...]], axis=1) * dinv[:, None]
    xn = x_ref[...] + jnp.maximum(g, 0.0)
    nrm = jnp.sqrt(jnp.sum(xn * xn, axis=1, keepdims=True))
    out_ref[...] = xn / jnp.maximum(nrm, 1e-12)


_dg_spec = pl.BlockSpec((NC, _RB, LANES), lambda i: (0, i, 0))
_x_spec = pl.BlockSpec((_RB, D), lambda i: (i, 0))
_h_spec = pl.BlockSpec((_RB, HALF), lambda i: (i, 0))
_w_spec = pl.BlockSpec((D, D), lambda i: (0, 0))


def _tc_pre(degp, x, w):
    return pl.pallas_call(
        _pre_body,
        grid=(_GRID,),
        in_specs=[_dg_spec, _x_spec, _w_spec],
        out_specs=[_h_spec, _h_spec],
        out_shape=[jax.ShapeDtypeStruct((NP, HALF), jnp.float32)] * 2,
    )(degp, x, w)


def _tc_mid(degp, x, a0, a1, w):
    return pl.pallas_call(
        _mid_body,
        grid=(_GRID,),
        in_specs=[_dg_spec, _x_spec, _h_spec, _h_spec, _w_spec],
        out_specs=[_x_spec, _h_spec, _h_spec],
        out_shape=[jax.ShapeDtypeStruct((NP, D), jnp.float32),
                   jax.ShapeDtypeStruct((NP, HALF), jnp.float32),
                   jax.ShapeDtypeStruct((NP, HALF), jnp.float32)],
    )(degp, x, a0, a1, w)


def _tc_post(degp, x, a0, a1):
    return pl.pallas_call(
        _post_body,
        grid=(_GRID,),
        in_specs=[_dg_spec, _x_spec, _h_spec, _h_spec],
        out_specs=_x_spec,
        out_shape=jax.ShapeDtypeStruct((NP, D), jnp.float32),
    )(degp, x, a0, a1)


def kernel(x, edge_index, W0, W1, W2):
    pad_e = N + (jnp.arange(E_PAD - E, dtype=jnp.int32) % (NP - N))
    src = jnp.concatenate([edge_index[0], pad_e])
    dst = jnp.concatenate([edge_index[1], pad_e])
    xp = jnp.pad(x, ((0, NP - N), (0, 0)))
    degp = _sc_degree(dst)
    m0, m1 = _tc_pre(degp, xp, W0)
    a0, a1 = _sc_spmm(src, dst, m0, m1)
    x1, m0, m1 = _tc_mid(degp, xp, a0, a1, W1)
    a0, a1 = _sc_spmm(src, dst, m0, m1)
    x2, m0, m1 = _tc_mid(degp, x1, a0, a1, W2)
    a0, a1 = _sc_spmm(src, dst, m0, m1)
    return _tc_post(degp, x2, a0, a1)[:N]
